# baseline (speedup 1.0000x reference)
import functools

import jax
import jax.numpy as jnp
from jax import lax
from jax.experimental import pallas as pl
from jax.experimental.pallas import tpu as pltpu
from jax.experimental.pallas import tpu_sc as plsc

N = 10000
IN = 128
HID = 128
OUT = 64
K = 10
ALPHA = 0.1
E = 320000

NT = 16
NPAD = 10240
ROWS_PT = NPAD // NT
CH = 128
NCH = ROWS_PT // CH
EB = 128
NBLK = 160
IBK = 80
NCHK = NBLK // IBK
NROWS = NBLK + 2
EPAD = NT * NROWS * EB
PADNODE = NPAD - 1

_F = 4


def _mlp_body(x_ref, w1_ref, b1_ref, w2_ref, b2_ref, o_ref):
    h = jnp.dot(x_ref[...], w1_ref[...], preferred_element_type=jnp.float32)
    h = jnp.maximum(h + b1_ref[...], 0.0)
    o_ref[...] = jnp.dot(h, w2_ref[...], preferred_element_type=jnp.float32) + b2_ref[...]


def _mlp(x, W1, b1, W2, b2):
    blk = 1000
    return pl.pallas_call(
        _mlp_body,
        grid=(N // blk,),
        in_specs=[
            pl.BlockSpec((blk, IN), lambda i: (i, 0)),
            pl.BlockSpec((IN, HID), lambda i: (0, 0)),
            pl.BlockSpec((1, HID), lambda i: (0, 0)),
            pl.BlockSpec((HID, OUT), lambda i: (0, 0)),
            pl.BlockSpec((1, OUT), lambda i: (0, 0)),
        ],
        out_specs=pl.BlockSpec((blk, OUT), lambda i: (i, 0)),
        out_shape=jax.ShapeDtypeStruct((N, OUT), jnp.float32),
    )(x, W1, b1, W2, b2)


def _rsqrt16(x):
    y = jnp.where(x >= 65536.0, x * 0.00390625,
                  jnp.where(x >= 256.0, x * 0.0625,
                            jnp.where(x >= 4.0, x * 0.5, x)))
    for _ in range(8):
        y = 0.5 * (y + x / y)
    return 1.0 / y


def _splat(ref1d, idx):
    return plsc.load_gather(ref1d, [jnp.full((16,), idx, jnp.int32)])


def _appnp_body(h0_hbm, src_hbm, dst_hbm, out_hbm, g_hbm,
                src_v, dst_v, rowsA, rowsB, accc, gc, h0c, zc, degl, dinvl,
                ones, gsA, gsB, ssA, ssB, ash, degsh):
    w = lax.axis_index("s")
    nbase = w * ROWS_PT

    def _stage_idx(k):
        pltpu.sync_copy(src_hbm.at[w, pl.ds(k * IBK, IBK + 2)], src_v)
        pltpu.sync_copy(dst_hbm.at[w, pl.ds(k * IBK, IBK + 2)], dst_v)

    def _fill_ones(i, c):
        ones[pl.ds(i * 16, 16)] = jnp.full((16,), 1.0, jnp.float32)
        return c
    lax.fori_loop(0, EB // 16, _fill_ones, 0)

    def _zero_dinvl(i, c):
        dinvl[pl.ds(i * 16, 16)] = jnp.zeros((16,), jnp.float32)
        return c
    lax.fori_loop(0, ROWS_PT // 16, _zero_dinvl, 0)
    pltpu.sync_copy(dinvl, degsh.at[pl.ds(nbase, ROWS_PT)])

    def _zero_zc(r, c):
        for f in range(_F):
            zc[r, pl.ds(f * 16, 16)] = jnp.zeros((16,), jnp.float32)
        return c
    lax.fori_loop(0, CH, _zero_zc, 0)

    def _zero_acc(c, carry):
        pltpu.sync_copy(zc, ash.at[pl.ds(nbase + c * CH, CH)])
        return carry
    lax.fori_loop(0, NCH, _zero_acc, 0)
    plsc.subcore_barrier()

    for k in range(NCHK):
        _stage_idx(k)

        def _deg_blk(j, cc):
            pltpu.sync_copy(ones, degsh.at[dst_v.at[j]], add=True)
            return cc
        lax.fori_loop(0, IBK, _deg_blk, 0)
    plsc.subcore_barrier()

    pltpu.sync_copy(degsh.at[pl.ds(nbase, ROWS_PT)], degl)

    def _scalars(i, c):
        s = pl.ds(i * 16, 16)
        d = degl[s] + 1.0
        degl[s] = d
        dinvl[s] = _rsqrt16(d)
        return c
    lax.fori_loop(0, ROWS_PT // 16, _scalars, 0)

    def _g_init(c, carry):
        base = nbase + c * CH
        pltpu.sync_copy(h0_hbm.at[pl.ds(base, CH)], h0c)

        def _row(r, cc):
            dv = _splat(dinvl, c * CH + r)
            for f in range(_F):
                s = pl.ds(f * 16, 16)
                gc[r, s] = dv * h0c[r, s]
            return cc
        lax.fori_loop(0, CH, _row, 0)
        pltpu.sync_copy(gc, g_hbm.at[pl.ds(base, CH)])
        return carry
    lax.fori_loop(0, NCH, _g_init, 0)
    plsc.subcore_barrier()

    def _gather_start(j, buf, sem):
        pltpu.async_copy(g_hbm.at[src_v.at[j]], buf, sem)

    def _gather_wait(j, buf, sem):
        pltpu.make_async_copy(g_hbm.at[src_v.at[j]], buf, sem).wait()

    def _scatter_start(j, buf, sem):
        pltpu.async_copy(buf, ash.at[dst_v.at[j]], sem, add=True)

    def _scatter_wait(j, buf, sem):
        pltpu.make_async_copy(buf, ash.at[dst_v.at[j]], sem).wait()

    def _step(t, carry):
        for k in range(NCHK):
            with jax.named_scope("stage_idx"):
                _stage_idx(k)

            def _blk(j, cc):
                pltpu.sync_copy(g_hbm.at[src_v.at[j]], rowsA)
                pltpu.sync_copy(rowsA, ash.at[dst_v.at[j]], add=True)
                return cc
            with jax.named_scope("edge_blocks"):
                lax.fori_loop(0, IBK, _blk, 0)
        with jax.named_scope("edge_barrier"):
            plsc.subcore_barrier()

        def _upd(c, cc):
            base = nbase + c * CH
            pltpu.sync_copy(ash.at[pl.ds(base, CH)], accc)
            pltpu.sync_copy(g_hbm.at[pl.ds(base, CH)], gc)
            pltpu.sync_copy(h0_hbm.at[pl.ds(base, CH)], h0c)
            pltpu.sync_copy(zc, ash.at[pl.ds(base, CH)])

            def _row(r, c3):
                dv = _splat(dinvl, c * CH + r)
                c2 = (1.0 - ALPHA) * dv * dv
                a2 = ALPHA * dv
                for f in range(_F):
                    s = pl.ds(f * 16, 16)
                    gc[r, s] = c2 * (accc[r, s] + gc[r, s]) + a2 * h0c[r, s]
                return c3
            lax.fori_loop(0, CH, _row, 0)
            pltpu.sync_copy(gc, g_hbm.at[pl.ds(base, CH)])
            return cc
        with jax.named_scope("node_update"):
            lax.fori_loop(0, NCH, _upd, 0)
        with jax.named_scope("upd_barrier"):
            plsc.subcore_barrier()
        return carry
    lax.fori_loop(0, K, _step, 0)

    def _out(c, carry):
        base = nbase + c * CH
        pltpu.sync_copy(g_hbm.at[pl.ds(base, CH)], gc)

        def _row(r, cc):
            dv = _splat(dinvl, c * CH + r)
            dp = _splat(degl, c * CH + r)
            s0 = dv * dp
            for f in range(_F):
                s = pl.ds(f * 16, 16)
                gc[r, s] = gc[r, s] * s0
            return cc
        lax.fori_loop(0, CH, _row, 0)
        pltpu.sync_copy(gc, out_hbm.at[pl.ds(base, CH)])
        return carry
    lax.fori_loop(0, NCH, _out, 0)


_appnp = functools.partial(
    pl.kernel,
    mesh=plsc.VectorSubcoreMesh(core_axis_name="c", subcore_axis_name="s",
                                num_cores=1),
    compiler_params=pltpu.CompilerParams(needs_layout_passes=False,
                                         use_tc_tiling_on_sc=False),
    out_type=(jax.ShapeDtypeStruct((NPAD, OUT), jnp.float32),
              jax.ShapeDtypeStruct((NPAD, OUT), jnp.float32)),
    scratch_types=[
        pltpu.VMEM((IBK + 2, EB), jnp.int32),
        pltpu.VMEM((IBK + 2, EB), jnp.int32),
        pltpu.VMEM((EB, OUT), jnp.float32),
        pltpu.VMEM((EB, OUT), jnp.float32),
        pltpu.VMEM((CH, OUT), jnp.float32),
        pltpu.VMEM((CH, OUT), jnp.float32),
        pltpu.VMEM((CH, OUT), jnp.float32),
        pltpu.VMEM((CH, OUT), jnp.float32),
        pltpu.VMEM((ROWS_PT,), jnp.float32),
        pltpu.VMEM((ROWS_PT,), jnp.float32),
        pltpu.VMEM((EB,), jnp.float32),
        pltpu.SemaphoreType.DMA,
        pltpu.SemaphoreType.DMA,
        pltpu.SemaphoreType.DMA,
        pltpu.SemaphoreType.DMA,
        pltpu.VMEM_SHARED((NPAD, OUT), jnp.float32),
        pltpu.VMEM_SHARED((NPAD,), jnp.float32),
    ],
)(_appnp_body)


def kernel(x, edge_index, W1, b1, W2, b2):
    h0 = _mlp(x, W1, b1.reshape(1, HID), W2, b2.reshape(1, OUT))
    h0p = jnp.zeros((NPAD, OUT), jnp.float32).at[:N].set(h0)
    padt = jnp.full((NT, NROWS * EB - E // NT), PADNODE, jnp.int32)
    src3 = jnp.concatenate([edge_index[0].reshape(NT, E // NT), padt],
                           axis=1).reshape(NT, NROWS, EB)
    dst3 = jnp.concatenate([edge_index[1].reshape(NT, E // NT), padt],
                           axis=1).reshape(NT, NROWS, EB)
    out, _ = _appnp(h0p, src3, dst3)
    return out[:N]


# speedup vs baseline: 1.4676x; 1.4676x over previous
import functools

import jax
import jax.numpy as jnp
from jax import lax
from jax.experimental import pallas as pl
from jax.experimental.pallas import tpu as pltpu
from jax.experimental.pallas import tpu_sc as plsc

N = 10000
IN = 128
HID = 128
OUT = 64
K = 10
ALPHA = 0.1
E = 320000

NT = 16
NPAD = 10240
ROWS_PT = NPAD // NT
CH = 128
NCH = ROWS_PT // CH
EB = 128
NBLK = 157
NROWS = NBLK
EPAD = NT * NROWS * EB
PADNODE = NPAD - 1

_F = 4


def _mlp_body(x_ref, w1_ref, b1_ref, w2_ref, b2_ref, o_ref):
    h = jnp.dot(x_ref[...], w1_ref[...], preferred_element_type=jnp.float32)
    h = jnp.maximum(h + b1_ref[...], 0.0)
    o_ref[...] = jnp.dot(h, w2_ref[...], preferred_element_type=jnp.float32) + b2_ref[...]


def _mlp(x, W1, b1, W2, b2):
    blk = 1000
    return pl.pallas_call(
        _mlp_body,
        grid=(N // blk,),
        in_specs=[
            pl.BlockSpec((blk, IN), lambda i: (i, 0)),
            pl.BlockSpec((IN, HID), lambda i: (0, 0)),
            pl.BlockSpec((1, HID), lambda i: (0, 0)),
            pl.BlockSpec((HID, OUT), lambda i: (0, 0)),
            pl.BlockSpec((1, OUT), lambda i: (0, 0)),
        ],
        out_specs=pl.BlockSpec((blk, OUT), lambda i: (i, 0)),
        out_shape=jax.ShapeDtypeStruct((N, OUT), jnp.float32),
    )(x, W1, b1, W2, b2)


def _rsqrt16(x):
    y = jnp.where(x >= 65536.0, x * 0.00390625,
                  jnp.where(x >= 256.0, x * 0.0625,
                            jnp.where(x >= 4.0, x * 0.5, x)))
    for _ in range(8):
        y = 0.5 * (y + x / y)
    return 1.0 / y


def _splat(ref1d, idx):
    return plsc.load_gather(ref1d, [jnp.full((16,), idx, jnp.int32)])


def _appnp_body(h0_hbm, src_hbm, dst_hbm, out_hbm, g_hbm,
                src_v, dst_v, rowsA, accc, gc, h0c, zc, degl, dinvl,
                ones, ash, degsh):
    w = lax.axis_index("s")
    nbase = w * ROWS_PT

    pltpu.sync_copy(src_hbm.at[w], src_v)
    pltpu.sync_copy(dst_hbm.at[w], dst_v)

    def _fill_ones(i, c):
        ones[pl.ds(i * 16, 16)] = jnp.full((16,), 1.0, jnp.float32)
        return c
    lax.fori_loop(0, EB // 16, _fill_ones, 0)

    def _zero_dinvl(i, c):
        dinvl[pl.ds(i * 16, 16)] = jnp.zeros((16,), jnp.float32)
        return c
    lax.fori_loop(0, ROWS_PT // 16, _zero_dinvl, 0)
    pltpu.sync_copy(dinvl, degsh.at[pl.ds(nbase, ROWS_PT)])

    def _zero_zc(r, c):
        for f in range(_F):
            zc[r, pl.ds(f * 16, 16)] = jnp.zeros((16,), jnp.float32)
        return c
    lax.fori_loop(0, CH, _zero_zc, 0)

    def _zero_acc(c, carry):
        pltpu.sync_copy(zc, ash.at[pl.ds(nbase + c * CH, CH)])
        return carry
    lax.fori_loop(0, NCH, _zero_acc, 0)
    plsc.subcore_barrier()

    def _deg_blk(j, cc):
        pltpu.sync_copy(ones, degsh.at[dst_v.at[j]], add=True)
        return cc
    lax.fori_loop(0, NBLK, _deg_blk, 0)
    plsc.subcore_barrier()

    pltpu.sync_copy(degsh.at[pl.ds(nbase, ROWS_PT)], degl)

    def _scalars(i, c):
        s = pl.ds(i * 16, 16)
        d = degl[s] + 1.0
        degl[s] = d
        dinvl[s] = _rsqrt16(d)
        return c
    lax.fori_loop(0, ROWS_PT // 16, _scalars, 0)

    def _g_init(c, carry):
        base = nbase + c * CH
        pltpu.sync_copy(h0_hbm.at[pl.ds(base, CH)], h0c)

        def _row(r, cc):
            dv = _splat(dinvl, c * CH + r)
            for f in range(_F):
                s = pl.ds(f * 16, 16)
                gc[r, s] = dv * h0c[r, s]
            return cc
        lax.fori_loop(0, CH, _row, 0)
        pltpu.sync_copy(gc, g_hbm.at[pl.ds(base, CH)])
        return carry
    lax.fori_loop(0, NCH, _g_init, 0)
    plsc.subcore_barrier()

    def _gather_start(j, buf, sem):
        pltpu.async_copy(g_hbm.at[src_v.at[j]], buf, sem)

    def _gather_wait(j, buf, sem):
        pltpu.make_async_copy(g_hbm.at[src_v.at[j]], buf, sem).wait()

    def _scatter_start(j, buf, sem):
        pltpu.async_copy(buf, ash.at[dst_v.at[j]], sem, add=True)

    def _scatter_wait(j, buf, sem):
        pltpu.make_async_copy(buf, ash.at[dst_v.at[j]], sem).wait()

    def _step(t, carry):
        def _blk(j, cc):
            pltpu.sync_copy(g_hbm.at[src_v.at[j]], rowsA)
            pltpu.sync_copy(rowsA, ash.at[dst_v.at[j]], add=True)
            return cc
        lax.fori_loop(0, NBLK, _blk, 0)
        plsc.subcore_barrier()

        def _upd(c, cc):
            base = nbase + c * CH
            pltpu.sync_copy(ash.at[pl.ds(base, CH)], accc)
            pltpu.sync_copy(g_hbm.at[pl.ds(base, CH)], gc)
            pltpu.sync_copy(h0_hbm.at[pl.ds(base, CH)], h0c)
            pltpu.sync_copy(zc, ash.at[pl.ds(base, CH)])

            def _row(r, c3):
                dv = _splat(dinvl, c * CH + r)
                c2 = (1.0 - ALPHA) * dv * dv
                a2 = ALPHA * dv
                for f in range(_F):
                    s = pl.ds(f * 16, 16)
                    gc[r, s] = c2 * (accc[r, s] + gc[r, s]) + a2 * h0c[r, s]
                return c3
            lax.fori_loop(0, CH, _row, 0)
            pltpu.sync_copy(gc, g_hbm.at[pl.ds(base, CH)])
            return cc
        lax.fori_loop(0, NCH, _upd, 0)
        plsc.subcore_barrier()
        return carry
    lax.fori_loop(0, K, _step, 0)

    def _out(c, carry):
        base = nbase + c * CH
        pltpu.sync_copy(g_hbm.at[pl.ds(base, CH)], gc)

        def _row(r, cc):
            dv = _splat(dinvl, c * CH + r)
            dp = _splat(degl, c * CH + r)
            s0 = dv * dp
            for f in range(_F):
                s = pl.ds(f * 16, 16)
                gc[r, s] = gc[r, s] * s0
            return cc
        lax.fori_loop(0, CH, _row, 0)
        pltpu.sync_copy(gc, out_hbm.at[pl.ds(base, CH)])
        return carry
    lax.fori_loop(0, NCH, _out, 0)


_appnp = functools.partial(
    pl.kernel,
    mesh=plsc.VectorSubcoreMesh(core_axis_name="c", subcore_axis_name="s",
                                num_cores=1),
    compiler_params=pltpu.CompilerParams(needs_layout_passes=False,
                                         use_tc_tiling_on_sc=False),
    out_type=(jax.ShapeDtypeStruct((NPAD, OUT), jnp.float32),
              jax.ShapeDtypeStruct((NPAD, OUT), jnp.float32)),
    scratch_types=[
        pltpu.VMEM((NROWS, EB), jnp.int32),
        pltpu.VMEM((NROWS, EB), jnp.int32),
        pltpu.VMEM((EB, OUT), jnp.float32),
        pltpu.VMEM((CH, OUT), jnp.float32),
        pltpu.VMEM((CH, OUT), jnp.float32),
        pltpu.VMEM((CH, OUT), jnp.float32),
        pltpu.VMEM((CH, OUT), jnp.float32),
        pltpu.VMEM((ROWS_PT,), jnp.float32),
        pltpu.VMEM((ROWS_PT,), jnp.float32),
        pltpu.VMEM((EB,), jnp.float32),
        pltpu.VMEM_SHARED((NPAD, OUT), jnp.float32),
        pltpu.VMEM_SHARED((NPAD,), jnp.float32),
    ],
)(_appnp_body)


def kernel(x, edge_index, W1, b1, W2, b2):
    h0 = _mlp(x, W1, b1.reshape(1, HID), W2, b2.reshape(1, OUT))
    h0p = jnp.zeros((NPAD, OUT), jnp.float32).at[:N].set(h0)
    padt = jnp.full((NT, NROWS * EB - E // NT), PADNODE, jnp.int32)
    src3 = jnp.concatenate([edge_index[0].reshape(NT, E // NT), padt],
                           axis=1).reshape(NT, NROWS, EB)
    dst3 = jnp.concatenate([edge_index[1].reshape(NT, E // NT), padt],
                           axis=1).reshape(NT, NROWS, EB)
    out, _ = _appnp(h0p, src3, dst3)
    return out[:N]


# speedup vs baseline: 1.9041x; 1.2974x over previous
import functools

import jax
import jax.numpy as jnp
from jax import lax
from jax.experimental import pallas as pl
from jax.experimental.pallas import tpu as pltpu
from jax.experimental.pallas import tpu_sc as plsc

N = 10000
IN = 128
HID = 128
OUT = 64
K = 10
ALPHA = 0.1
E = 320000

NT = 16
NPAD = 10240
ROWS_PT = NPAD // NT
CH = 128
NCH = ROWS_PT // CH
EB = 128
NBLK = 157
NROWS = NBLK
EPAD = NT * NROWS * EB
PADNODE = NPAD - 1

_F = 4


def _mlp_body(x_ref, w1_ref, b1_ref, w2_ref, b2_ref, o_ref):
    h = jnp.dot(x_ref[...], w1_ref[...], preferred_element_type=jnp.float32)
    h = jnp.maximum(h + b1_ref[...], 0.0)
    o_ref[...] = jnp.dot(h, w2_ref[...], preferred_element_type=jnp.float32) + b2_ref[...]


def _mlp(x, W1, b1, W2, b2):
    blk = 1000
    return pl.pallas_call(
        _mlp_body,
        grid=(N // blk,),
        in_specs=[
            pl.BlockSpec((blk, IN), lambda i: (i, 0)),
            pl.BlockSpec((IN, HID), lambda i: (0, 0)),
            pl.BlockSpec((1, HID), lambda i: (0, 0)),
            pl.BlockSpec((HID, OUT), lambda i: (0, 0)),
            pl.BlockSpec((1, OUT), lambda i: (0, 0)),
        ],
        out_specs=pl.BlockSpec((blk, OUT), lambda i: (i, 0)),
        out_shape=jax.ShapeDtypeStruct((N, OUT), jnp.float32),
    )(x, W1, b1, W2, b2)


def _rsqrt16(x):
    y = jnp.where(x >= 65536.0, x * 0.00390625,
                  jnp.where(x >= 256.0, x * 0.0625,
                            jnp.where(x >= 4.0, x * 0.5, x)))
    for _ in range(8):
        y = 0.5 * (y + x / y)
    return 1.0 / y


def _splat(ref1d, idx):
    return plsc.load_gather(ref1d, [jnp.full((16,), idx, jnp.int32)])


def _appnp_body(h0_hbm, src_hbm, dst_hbm, out_hbm, g_hbm,
                src_v, dst_v, rowsA, accc, gc, h0c, zc, degl, dinvl,
                ones, gsA, gsB, ssA, ssB, ash, degsh):
    w = lax.axis_index("s")
    nbase = w * ROWS_PT

    pltpu.sync_copy(src_hbm.at[w], src_v)
    pltpu.sync_copy(dst_hbm.at[w], dst_v)

    def _fill_ones(i, c):
        ones[pl.ds(i * 16, 16)] = jnp.full((16,), 1.0, jnp.float32)
        return c
    lax.fori_loop(0, EB // 16, _fill_ones, 0)

    def _zero_dinvl(i, c):
        dinvl[pl.ds(i * 16, 16)] = jnp.zeros((16,), jnp.float32)
        return c
    lax.fori_loop(0, ROWS_PT // 16, _zero_dinvl, 0)
    pltpu.sync_copy(dinvl, degsh.at[pl.ds(nbase, ROWS_PT)])

    def _zero_zc(r, c):
        for f in range(_F):
            zc[r, pl.ds(f * 16, 16)] = jnp.zeros((16,), jnp.float32)
        return c
    lax.fori_loop(0, CH, _zero_zc, 0)

    def _zero_acc(c, carry):
        pltpu.sync_copy(zc, ash.at[pl.ds(nbase + c * CH, CH)])
        return carry
    lax.fori_loop(0, NCH, _zero_acc, 0)
    plsc.subcore_barrier()

    def _deg_blk(j, cc):
        pltpu.sync_copy(ones, degsh.at[dst_v.at[j]], add=True)
        return cc
    lax.fori_loop(0, NBLK, _deg_blk, 0)
    plsc.subcore_barrier()

    pltpu.sync_copy(degsh.at[pl.ds(nbase, ROWS_PT)], degl)

    def _scalars(i, c):
        s = pl.ds(i * 16, 16)
        d = degl[s] + 1.0
        degl[s] = d
        dinvl[s] = _rsqrt16(d)
        return c
    lax.fori_loop(0, ROWS_PT // 16, _scalars, 0)

    def _g_init(c, carry):
        base = nbase + c * CH
        pltpu.sync_copy(h0_hbm.at[pl.ds(base, CH)], h0c)

        def _row(r, cc):
            dv = _splat(dinvl, c * CH + r)
            for f in range(_F):
                s = pl.ds(f * 16, 16)
                gc[r, s] = dv * h0c[r, s]
            return cc
        lax.fori_loop(0, CH, _row, 0)
        pltpu.sync_copy(gc, g_hbm.at[pl.ds(base, CH)])
        return carry
    lax.fori_loop(0, NCH, _g_init, 0)
    plsc.subcore_barrier()

    def _gather_start(j, buf, sem):
        pltpu.async_copy(g_hbm.at[src_v.at[j]], buf, sem)

    def _gather_wait(j, buf, sem):
        pltpu.make_async_copy(g_hbm.at[src_v.at[j]], buf, sem).wait()

    def _scatter_start(j, buf, sem):
        pltpu.async_copy(buf, ash.at[dst_v.at[j]], sem, add=True)

    def _scatter_wait(j, buf, sem):
        pltpu.make_async_copy(buf, ash.at[dst_v.at[j]], sem).wait()

    def _gather_start(j, buf, sem):
        pltpu.async_copy(g_hbm.at[src_v.at[j]], buf, sem)

    def _gather_wait(j, buf, sem):
        pltpu.make_async_copy(g_hbm.at[src_v.at[j]], buf, sem).wait()

    def _scatter_start(j, buf, sem):
        pltpu.async_copy(buf, ash.at[dst_v.at[j]], sem, add=True)

    def _scatter_wait(j, buf, sem):
        pltpu.make_async_copy(buf, ash.at[dst_v.at[j]], sem).wait()

    def _step(t, carry):
        _gather_start(0, rowsA, gsA)
        _gather_start(1, accc, gsB)

        def _pair(pp, cc):
            j0 = 2 * pp
            j1 = j0 + 1
            _gather_wait(j0, rowsA, gsA)
            _scatter_start(j0, rowsA, ssA)
            _gather_wait(j1, accc, gsB)
            _scatter_start(j1, accc, ssB)
            _scatter_wait(j0, rowsA, ssA)
            _gather_start(jnp.minimum(j0 + 2, NBLK - 2), rowsA, gsA)
            _scatter_wait(j1, accc, ssB)
            _gather_start(jnp.minimum(j1 + 2, NBLK - 2), accc, gsB)
            return cc
        lax.fori_loop(0, (NBLK - 1) // 2, _pair, 0)
        _gather_wait(NBLK - 2, rowsA, gsA)
        _gather_wait(NBLK - 2, accc, gsB)
        pltpu.sync_copy(g_hbm.at[src_v.at[NBLK - 1]], rowsA)
        pltpu.sync_copy(rowsA, ash.at[dst_v.at[NBLK - 1]], add=True)
        plsc.subcore_barrier()

        def _upd(c, cc):
            base = nbase + c * CH
            pltpu.sync_copy(ash.at[pl.ds(base, CH)], accc)
            pltpu.sync_copy(g_hbm.at[pl.ds(base, CH)], gc)
            pltpu.sync_copy(h0_hbm.at[pl.ds(base, CH)], h0c)
            pltpu.sync_copy(zc, ash.at[pl.ds(base, CH)])

            def _row(r, c3):
                dv = _splat(dinvl, c * CH + r)
                c2 = (1.0 - ALPHA) * dv * dv
                a2 = ALPHA * dv
                for f in range(_F):
                    s = pl.ds(f * 16, 16)
                    gc[r, s] = c2 * (accc[r, s] + gc[r, s]) + a2 * h0c[r, s]
                return c3
            lax.fori_loop(0, CH, _row, 0)
            pltpu.sync_copy(gc, g_hbm.at[pl.ds(base, CH)])
            return cc
        lax.fori_loop(0, NCH, _upd, 0)
        plsc.subcore_barrier()
        return carry
    lax.fori_loop(0, K, _step, 0)

    def _out(c, carry):
        base = nbase + c * CH
        pltpu.sync_copy(g_hbm.at[pl.ds(base, CH)], gc)

        def _row(r, cc):
            dv = _splat(dinvl, c * CH + r)
            dp = _splat(degl, c * CH + r)
            s0 = dv * dp
            for f in range(_F):
                s = pl.ds(f * 16, 16)
                gc[r, s] = gc[r, s] * s0
            return cc
        lax.fori_loop(0, CH, _row, 0)
        pltpu.sync_copy(gc, out_hbm.at[pl.ds(base, CH)])
        return carry
    lax.fori_loop(0, NCH, _out, 0)


_appnp = functools.partial(
    pl.kernel,
    mesh=plsc.VectorSubcoreMesh(core_axis_name="c", subcore_axis_name="s",
                                num_cores=1),
    compiler_params=pltpu.CompilerParams(needs_layout_passes=False,
                                         use_tc_tiling_on_sc=False),
    out_type=(jax.ShapeDtypeStruct((NPAD, OUT), jnp.float32),
              jax.ShapeDtypeStruct((NPAD, OUT), jnp.float32)),
    scratch_types=[
        pltpu.VMEM((NROWS, EB), jnp.int32),
        pltpu.VMEM((NROWS, EB), jnp.int32),
        pltpu.VMEM((EB, OUT), jnp.float32),
        pltpu.VMEM((CH, OUT), jnp.float32),
        pltpu.VMEM((CH, OUT), jnp.float32),
        pltpu.VMEM((CH, OUT), jnp.float32),
        pltpu.VMEM((CH, OUT), jnp.float32),
        pltpu.VMEM((ROWS_PT,), jnp.float32),
        pltpu.VMEM((ROWS_PT,), jnp.float32),
        pltpu.VMEM((EB,), jnp.float32),
        pltpu.SemaphoreType.DMA,
        pltpu.SemaphoreType.DMA,
        pltpu.SemaphoreType.DMA,
        pltpu.SemaphoreType.DMA,
        pltpu.VMEM_SHARED((NPAD, OUT), jnp.float32),
        pltpu.VMEM_SHARED((NPAD,), jnp.float32),
    ],
)(_appnp_body)


def kernel(x, edge_index, W1, b1, W2, b2):
    h0 = _mlp(x, W1, b1.reshape(1, HID), W2, b2.reshape(1, OUT))
    h0p = jnp.zeros((NPAD, OUT), jnp.float32).at[:N].set(h0)
    padt = jnp.full((NT, NROWS * EB - E // NT), PADNODE, jnp.int32)
    src3 = jnp.concatenate([edge_index[0].reshape(NT, E // NT), padt],
                           axis=1).reshape(NT, NROWS, EB)
    dst3 = jnp.concatenate([edge_index[1].reshape(NT, E // NT), padt],
                           axis=1).reshape(NT, NROWS, EB)
    out, _ = _appnp(h0p, src3, dst3)
    return out[:N]


# speedup vs baseline: 2.4851x; 1.3051x over previous
import functools

import jax
import jax.numpy as jnp
from jax import lax
from jax.experimental import pallas as pl
from jax.experimental.pallas import tpu as pltpu
from jax.experimental.pallas import tpu_sc as plsc

N = 10000
IN = 128
HID = 128
OUT = 64
K = 10
ALPHA = 0.1
E = 320000

NT = 16
NPAD = 10240
ROWS_PT = NPAD // NT
CH = 128
NCH = ROWS_PT // CH
EB = 128
NBLK = 157
NROWS = NBLK
EPAD = NT * NROWS * EB
PADNODE = NPAD - 1

_F = 4


def _mlp_body(x_ref, w1_ref, b1_ref, w2_ref, b2_ref, o_ref):
    h = jnp.dot(x_ref[...], w1_ref[...], preferred_element_type=jnp.float32)
    h = jnp.maximum(h + b1_ref[...], 0.0)
    o_ref[...] = jnp.dot(h, w2_ref[...], preferred_element_type=jnp.float32) + b2_ref[...]


def _mlp(x, W1, b1, W2, b2):
    blk = 1000
    return pl.pallas_call(
        _mlp_body,
        grid=(N // blk,),
        in_specs=[
            pl.BlockSpec((blk, IN), lambda i: (i, 0)),
            pl.BlockSpec((IN, HID), lambda i: (0, 0)),
            pl.BlockSpec((1, HID), lambda i: (0, 0)),
            pl.BlockSpec((HID, OUT), lambda i: (0, 0)),
            pl.BlockSpec((1, OUT), lambda i: (0, 0)),
        ],
        out_specs=pl.BlockSpec((blk, OUT), lambda i: (i, 0)),
        out_shape=jax.ShapeDtypeStruct((N, OUT), jnp.float32),
    )(x, W1, b1, W2, b2)


def _rsqrt16(x):
    y = jnp.where(x >= 65536.0, x * 0.00390625,
                  jnp.where(x >= 256.0, x * 0.0625,
                            jnp.where(x >= 4.0, x * 0.5, x)))
    for _ in range(8):
        y = 0.5 * (y + x / y)
    return 1.0 / y


def _splat(ref1d, idx):
    return plsc.load_gather(ref1d, [jnp.full((16,), idx, jnp.int32)])


def _appnp_body(h0_hbm, src_hbm, dst_hbm, out_hbm, g_hbm,
                src_v, dst_v, rowsA, accc, gc, h0c, zc, degl, dinvl,
                ones, gsA, gsB, gsC, gsD, ssA, ssB, ssC, ssD, ash, degsh):
    w = lax.axis_index("s")
    nbase = w * ROWS_PT

    pltpu.sync_copy(src_hbm.at[w], src_v)
    pltpu.sync_copy(dst_hbm.at[w], dst_v)

    def _fill_ones(i, c):
        ones[pl.ds(i * 16, 16)] = jnp.full((16,), 1.0, jnp.float32)
        return c
    lax.fori_loop(0, EB // 16, _fill_ones, 0)

    def _zero_dinvl(i, c):
        dinvl[pl.ds(i * 16, 16)] = jnp.zeros((16,), jnp.float32)
        return c
    lax.fori_loop(0, ROWS_PT // 16, _zero_dinvl, 0)
    pltpu.sync_copy(dinvl, degsh.at[pl.ds(nbase, ROWS_PT)])

    def _zero_zc(r, c):
        for f in range(_F):
            zc[r, pl.ds(f * 16, 16)] = jnp.zeros((16,), jnp.float32)
        return c
    lax.fori_loop(0, CH, _zero_zc, 0)

    def _zero_acc(c, carry):
        pltpu.sync_copy(zc, ash.at[pl.ds(nbase + c * CH, CH)])
        return carry
    lax.fori_loop(0, NCH, _zero_acc, 0)
    plsc.subcore_barrier()

    def _deg_blk(j, cc):
        pltpu.sync_copy(ones, degsh.at[dst_v.at[j]], add=True)
        return cc
    lax.fori_loop(0, NBLK, _deg_blk, 0)
    plsc.subcore_barrier()

    pltpu.sync_copy(degsh.at[pl.ds(nbase, ROWS_PT)], degl)

    def _scalars(i, c):
        s = pl.ds(i * 16, 16)
        d = degl[s] + 1.0
        degl[s] = d
        dinvl[s] = _rsqrt16(d)
        return c
    lax.fori_loop(0, ROWS_PT // 16, _scalars, 0)

    def _g_init(c, carry):
        base = nbase + c * CH
        pltpu.sync_copy(h0_hbm.at[pl.ds(base, CH)], h0c)

        def _row(r, cc):
            dv = _splat(dinvl, c * CH + r)
            for f in range(_F):
                s = pl.ds(f * 16, 16)
                gc[r, s] = dv * h0c[r, s]
            return cc
        lax.fori_loop(0, CH, _row, 0)
        pltpu.sync_copy(gc, g_hbm.at[pl.ds(base, CH)])
        return carry
    lax.fori_loop(0, NCH, _g_init, 0)
    plsc.subcore_barrier()

    def _gather_start(j, buf, sem):
        pltpu.async_copy(g_hbm.at[src_v.at[j]], buf, sem)

    def _gather_wait(j, buf, sem):
        pltpu.make_async_copy(g_hbm.at[src_v.at[j]], buf, sem).wait()

    def _scatter_start(j, buf, sem):
        pltpu.async_copy(buf, ash.at[dst_v.at[j]], sem, add=True)

    def _scatter_wait(j, buf, sem):
        pltpu.make_async_copy(buf, ash.at[dst_v.at[j]], sem).wait()

    def _gather_start(j, buf, sem):
        pltpu.async_copy(g_hbm.at[src_v.at[j]], buf, sem)

    def _gather_wait(j, buf, sem):
        pltpu.make_async_copy(g_hbm.at[src_v.at[j]], buf, sem).wait()

    def _scatter_start(j, buf, sem):
        pltpu.async_copy(buf, ash.at[dst_v.at[j]], sem, add=True)

    def _scatter_wait(j, buf, sem):
        pltpu.make_async_copy(buf, ash.at[dst_v.at[j]], sem).wait()

    ebufs = (rowsA, accc, gc, h0c)
    gsems = (gsA, gsB, gsC, gsD)
    ssems = (ssA, ssB, ssC, ssD)
    NQ = (NBLK - 1) // 4

    def _step(t, carry):
        for b in range(4):
            _gather_start(b, ebufs[b], gsems[b])

        def _quad(q, cc):
            j = 4 * q
            for b in range(4):
                _gather_wait(j + b, ebufs[b], gsems[b])
                _scatter_start(j + b, ebufs[b], ssems[b])
            for b in range(4):
                _scatter_wait(j + b, ebufs[b], ssems[b])
                _gather_start(jnp.minimum(j + b + 4, 4 * NQ - 4), ebufs[b],
                              gsems[b])
            return cc
        lax.fori_loop(0, NQ, _quad, 0)
        for b in range(4):
            _gather_wait(4 * NQ - 4, ebufs[b], gsems[b])
        for j in range(4 * NQ, NBLK):
            pltpu.sync_copy(g_hbm.at[src_v.at[j]], rowsA)
            pltpu.sync_copy(rowsA, ash.at[dst_v.at[j]], add=True)
        plsc.subcore_barrier()

        def _upd(c, cc):
            base = nbase + c * CH
            pltpu.sync_copy(ash.at[pl.ds(base, CH)], accc)
            pltpu.sync_copy(g_hbm.at[pl.ds(base, CH)], gc)
            pltpu.sync_copy(h0_hbm.at[pl.ds(base, CH)], h0c)
            pltpu.sync_copy(zc, ash.at[pl.ds(base, CH)])

            def _row(r, c3):
                dv = _splat(dinvl, c * CH + r)
                c2 = (1.0 - ALPHA) * dv * dv
                a2 = ALPHA * dv
                for f in range(_F):
                    s = pl.ds(f * 16, 16)
                    gc[r, s] = c2 * (accc[r, s] + gc[r, s]) + a2 * h0c[r, s]
                return c3
            lax.fori_loop(0, CH, _row, 0)
            pltpu.sync_copy(gc, g_hbm.at[pl.ds(base, CH)])
            return cc
        lax.fori_loop(0, NCH, _upd, 0)
        plsc.subcore_barrier()
        return carry
    lax.fori_loop(0, K, _step, 0)

    def _out(c, carry):
        base = nbase + c * CH
        pltpu.sync_copy(g_hbm.at[pl.ds(base, CH)], gc)

        def _row(r, cc):
            dv = _splat(dinvl, c * CH + r)
            dp = _splat(degl, c * CH + r)
            s0 = dv * dp
            for f in range(_F):
                s = pl.ds(f * 16, 16)
                gc[r, s] = gc[r, s] * s0
            return cc
        lax.fori_loop(0, CH, _row, 0)
        pltpu.sync_copy(gc, out_hbm.at[pl.ds(base, CH)])
        return carry
    lax.fori_loop(0, NCH, _out, 0)


_appnp = functools.partial(
    pl.kernel,
    mesh=plsc.VectorSubcoreMesh(core_axis_name="c", subcore_axis_name="s",
                                num_cores=1),
    compiler_params=pltpu.CompilerParams(needs_layout_passes=False,
                                         use_tc_tiling_on_sc=False),
    out_type=(jax.ShapeDtypeStruct((NPAD, OUT), jnp.float32),
              jax.ShapeDtypeStruct((NPAD, OUT), jnp.float32)),
    scratch_types=[
        pltpu.VMEM((NROWS, EB), jnp.int32),
        pltpu.VMEM((NROWS, EB), jnp.int32),
        pltpu.VMEM((EB, OUT), jnp.float32),
        pltpu.VMEM((CH, OUT), jnp.float32),
        pltpu.VMEM((CH, OUT), jnp.float32),
        pltpu.VMEM((CH, OUT), jnp.float32),
        pltpu.VMEM((CH, OUT), jnp.float32),
        pltpu.VMEM((ROWS_PT,), jnp.float32),
        pltpu.VMEM((ROWS_PT,), jnp.float32),
        pltpu.VMEM((EB,), jnp.float32),
        pltpu.SemaphoreType.DMA,
        pltpu.SemaphoreType.DMA,
        pltpu.SemaphoreType.DMA,
        pltpu.SemaphoreType.DMA,
        pltpu.SemaphoreType.DMA,
        pltpu.SemaphoreType.DMA,
        pltpu.SemaphoreType.DMA,
        pltpu.SemaphoreType.DMA,
        pltpu.VMEM_SHARED((NPAD, OUT), jnp.float32),
        pltpu.VMEM_SHARED((NPAD,), jnp.float32),
    ],
)(_appnp_body)


def kernel(x, edge_index, W1, b1, W2, b2):
    h0 = _mlp(x, W1, b1.reshape(1, HID), W2, b2.reshape(1, OUT))
    h0p = jnp.zeros((NPAD, OUT), jnp.float32).at[:N].set(h0)
    padt = jnp.full((NT, NROWS * EB - E // NT), PADNODE, jnp.int32)
    src3 = jnp.concatenate([edge_index[0].reshape(NT, E // NT), padt],
                           axis=1).reshape(NT, NROWS, EB)
    dst3 = jnp.concatenate([edge_index[1].reshape(NT, E // NT), padt],
                           axis=1).reshape(NT, NROWS, EB)
    out, _ = _appnp(h0p, src3, dst3)
    return out[:N]
